# speedup vs baseline: 2.0943x; 2.0943x over previous
import functools

import jax
import jax.numpy as jnp
from jax import lax
from jax.experimental import pallas as pl
from jax.experimental.pallas import tpu as pltpu
from jax.experimental.pallas import tpu_sc as plsc

B = 25000
D = 128
S = 20
NV = D // 16
NC = 2
NS = 16
NW = NC * NS
BPW = 800
B_PAD = NW * BPW
Q = 16
NCH = BPW // Q
G = 4
GROWS = Q * S // G

_mesh = plsc.VectorSubcoreMesh(core_axis_name="c", subcore_axis_name="s")


@functools.partial(
    pl.kernel,
    out_type=jax.ShapeDtypeStruct((B_PAD, 2 * D), jnp.float32),
    mesh=_mesh,
    scratch_types=[
        pltpu.VMEM((BPW,), jnp.int32),
        pltpu.VMEM((BPW * S,), jnp.int32),
        pltpu.VMEM((Q * S, D), jnp.float32),
        pltpu.VMEM((Q * S, D), jnp.float32),
        pltpu.VMEM((Q, D), jnp.float32),
        pltpu.VMEM((Q, D), jnp.float32),
        pltpu.VMEM((Q, 2 * D), jnp.float32),
        pltpu.VMEM((Q, 2 * D), jnp.float32),
        pltpu.SemaphoreType.DMA,
        pltpu.SemaphoreType.DMA,
        pltpu.SemaphoreType.DMA,
        pltpu.SemaphoreType.DMA,
    ],
)
def _sc_gather(feat_hbm, nodes_hbm, neigh_hbm, comb_hbm,
               nodes_v, nidx_v, rows0, rows1, srows0, srows1,
               stage0, stage1, gsem0, gsem1, osem0, osem1):
    wid = lax.axis_index("s") * NC + lax.axis_index("c")
    base = wid * BPW
    pltpu.sync_copy(nodes_hbm.at[pl.ds(base, BPW)], nodes_v)
    pltpu.sync_copy(neigh_hbm.at[pl.ds(base * S, BPW * S)], nidx_v)

    rows = (rows0, rows1)
    srows = (srows0, srows1)
    stage = (stage0, stage1)
    gsem = (gsem0, gsem1)
    osem = (osem0, osem1)

    def issue(c, p):
        for g in range(G):
            pltpu.async_copy(
                feat_hbm.at[nidx_v.at[pl.ds(c * (Q * S) + g * GROWS, GROWS)]],
                rows[p].at[pl.ds(g * GROWS, GROWS)],
                gsem[p])
        pltpu.async_copy(feat_hbm.at[nodes_v.at[pl.ds(c * Q, Q)]],
                         srows[p], gsem[p])

    def wait_gathers(p):
        for g in range(G):
            pltpu.make_async_copy(
                feat_hbm.at[pl.ds(0, GROWS)],
                rows[p].at[pl.ds(g * GROWS, GROWS)],
                gsem[p]).wait()
        pltpu.make_async_copy(feat_hbm.at[pl.ds(0, Q)], srows[p],
                              gsem[p]).wait()

    def accum(p):
        r = rows[p]
        sr = srows[p]
        st = stage[p]

        def qbody(q, carry):
            for v in range(NV):
                sl = pl.ds(v * 16, 16)
                st[q, sl] = sr[q, sl]
            accs = [r[q * S, pl.ds(v * 16, 16)] for v in range(NV)]
            for s in range(1, S):
                for v in range(NV):
                    accs[v] = accs[v] + r[q * S + s, pl.ds(v * 16, 16)]
            for v in range(NV):
                st[q, pl.ds(D + v * 16, 16)] = accs[v] * (1.0 / S)
            return carry

        lax.fori_loop(0, Q, qbody, 0)

    issue(0, 0)
    issue(1, 1)

    def tbody(t, carry):
        for p in (0, 1):
            c = 2 * t + p
            wait_gathers(p)

            @pl.when(t > 0)
            def _wait_out():
                pltpu.make_async_copy(stage[p], comb_hbm.at[pl.ds(0, Q)],
                                      osem[p]).wait()

            accum(p)
            pltpu.async_copy(stage[p], comb_hbm.at[pl.ds(base + c * Q, Q)],
                             osem[p])
            cn = c + 2

            @pl.when(cn < NCH)
            def _issue_next():
                issue(cn, p)
        return carry

    lax.fori_loop(0, NCH // 2, tbody, 0)
    for p in (0, 1):
        pltpu.make_async_copy(stage[p], comb_hbm.at[pl.ds(0, Q)],
                              osem[p]).wait()


BLK = 1600


def _mm_body(x_ref, w_ref, o_ref):
    o_ref[...] = jnp.maximum(
        jnp.dot(x_ref[...], w_ref[...], preferred_element_type=jnp.float32),
        0.0)


def _tc_matmul(comb, w):
    return pl.pallas_call(
        _mm_body,
        grid=(B_PAD // BLK,),
        in_specs=[
            pl.BlockSpec((BLK, 2 * D), lambda i: (i, 0)),
            pl.BlockSpec((2 * D, D), lambda i: (0, 0)),
        ],
        out_specs=pl.BlockSpec((BLK, D), lambda i: (i, 0)),
        out_shape=jax.ShapeDtypeStruct((B_PAD, D), jnp.float32),
    )(comb, w)


@jax.jit
def kernel(nodes, neigh_idx, features, weight):
    nodes_i = nodes.astype(jnp.int32)
    neigh_i = neigh_idx.astype(jnp.int32)
    nodes_p = jnp.pad(nodes_i, (0, B_PAD - B))
    neigh_p = jnp.pad(neigh_i, ((0, B_PAD - B), (0, 0))).reshape(-1)
    comb = _sc_gather(features, nodes_p, neigh_p)
    out = _tc_matmul(comb, weight)
    return out[:B]


# speedup vs baseline: 2.1505x; 1.0269x over previous
import functools

import jax
import jax.numpy as jnp
from jax import lax
from jax.experimental import pallas as pl
from jax.experimental.pallas import tpu as pltpu
from jax.experimental.pallas import tpu_sc as plsc

B = 25000
D = 128
S = 20
NV = D // 16
NC = 2
NS = 16
NW = NC * NS
B_PAD = 25600
Q = 16
G = 4
GROWS = Q * S // G

BPW0 = 1344
BPW1 = 256
NCH0 = BPW0 // Q
NCH1 = BPW1 // Q
assert NS * (BPW0 + BPW1) == B_PAD
CORE1_BASE = NS * BPW0

_mesh = plsc.VectorSubcoreMesh(core_axis_name="c", subcore_axis_name="s")


@functools.partial(
    pl.kernel,
    out_type=jax.ShapeDtypeStruct((B_PAD, 2 * D), jnp.float32),
    mesh=_mesh,
    scratch_types=[
        pltpu.VMEM((BPW0,), jnp.int32),
        pltpu.VMEM((BPW0 * S,), jnp.int32),
        pltpu.VMEM((Q * S, D), jnp.float32),
        pltpu.VMEM((Q * S, D), jnp.float32),
        pltpu.VMEM((Q, D), jnp.float32),
        pltpu.VMEM((Q, D), jnp.float32),
        pltpu.VMEM((Q, 2 * D), jnp.float32),
        pltpu.VMEM((Q, 2 * D), jnp.float32),
        pltpu.SemaphoreType.DMA,
        pltpu.SemaphoreType.DMA,
        pltpu.SemaphoreType.DMA,
        pltpu.SemaphoreType.DMA,
    ],
)
def _sc_gather(feat_hbm, nodes_hbm, neigh_hbm, comb_hbm,
               nodes_v, nidx_v, rows0, rows1, srows0, srows1,
               stage0, stage1, gsem0, gsem1, osem0, osem1):
    cid = lax.axis_index("c")
    sid = lax.axis_index("s")
    base = lax.select(cid == 0, sid * BPW0, CORE1_BASE + sid * BPW1)
    nch = lax.select(cid == 0, NCH0, NCH1)

    @pl.when(cid == 0)
    def _load_idx0():
        pltpu.sync_copy(nodes_hbm.at[pl.ds(base, BPW0)], nodes_v)
        pltpu.sync_copy(neigh_hbm.at[pl.ds(base * S, BPW0 * S)], nidx_v)

    @pl.when(cid != 0)
    def _load_idx1():
        pltpu.sync_copy(nodes_hbm.at[pl.ds(base, BPW1)],
                        nodes_v.at[pl.ds(0, BPW1)])
        pltpu.sync_copy(neigh_hbm.at[pl.ds(base * S, BPW1 * S)],
                        nidx_v.at[pl.ds(0, BPW1 * S)])

    rows = (rows0, rows1)
    srows = (srows0, srows1)
    stage = (stage0, stage1)
    gsem = (gsem0, gsem1)
    osem = (osem0, osem1)

    def issue(c, p):
        for g in range(G):
            pltpu.async_copy(
                feat_hbm.at[nidx_v.at[pl.ds(c * (Q * S) + g * GROWS, GROWS)]],
                rows[p].at[pl.ds(g * GROWS, GROWS)],
                gsem[p])
        pltpu.async_copy(feat_hbm.at[nodes_v.at[pl.ds(c * Q, Q)]],
                         srows[p], gsem[p])

    def wait_gathers(p):
        for g in range(G):
            pltpu.make_async_copy(
                feat_hbm.at[pl.ds(0, GROWS)],
                rows[p].at[pl.ds(g * GROWS, GROWS)],
                gsem[p]).wait()
        pltpu.make_async_copy(feat_hbm.at[pl.ds(0, Q)], srows[p],
                              gsem[p]).wait()

    def accum(p):
        r = rows[p]
        sr = srows[p]
        st = stage[p]

        def qbody(q, carry):
            for v in range(NV):
                sl = pl.ds(v * 16, 16)
                st[q, sl] = sr[q, sl]
            accs = [r[q * S, pl.ds(v * 16, 16)] for v in range(NV)]
            for s in range(1, S):
                for v in range(NV):
                    accs[v] = accs[v] + r[q * S + s, pl.ds(v * 16, 16)]
            for v in range(NV):
                st[q, pl.ds(D + v * 16, 16)] = accs[v] * (1.0 / S)
            return carry

        lax.fori_loop(0, Q, qbody, 0)

    issue(0, 0)
    issue(1, 1)

    def tbody(t, carry):
        for p in (0, 1):
            c = 2 * t + p
            wait_gathers(p)

            @pl.when(t > 0)
            def _wait_out():
                pltpu.make_async_copy(stage[p], comb_hbm.at[pl.ds(0, Q)],
                                      osem[p]).wait()

            accum(p)
            pltpu.async_copy(stage[p], comb_hbm.at[pl.ds(base + c * Q, Q)],
                             osem[p])
            cn = c + 2

            @pl.when(cn < nch)
            def _issue_next():
                issue(cn, p)
        return carry

    lax.fori_loop(0, nch // 2, tbody, 0)
    for p in (0, 1):
        pltpu.make_async_copy(stage[p], comb_hbm.at[pl.ds(0, Q)],
                              osem[p]).wait()


BLK = 1600


def _mm_body(x_ref, w_ref, o_ref):
    o_ref[...] = jnp.maximum(
        jnp.dot(x_ref[...], w_ref[...], preferred_element_type=jnp.float32),
        0.0)


def _tc_matmul(comb, w):
    return pl.pallas_call(
        _mm_body,
        grid=(B_PAD // BLK,),
        in_specs=[
            pl.BlockSpec((BLK, 2 * D), lambda i: (i, 0)),
            pl.BlockSpec((2 * D, D), lambda i: (0, 0)),
        ],
        out_specs=pl.BlockSpec((BLK, D), lambda i: (i, 0)),
        out_shape=jax.ShapeDtypeStruct((B_PAD, D), jnp.float32),
    )(comb, w)


@jax.jit
def kernel(nodes, neigh_idx, features, weight):
    nodes_i = nodes.astype(jnp.int32)
    neigh_i = neigh_idx.astype(jnp.int32)
    nodes_p = jnp.pad(nodes_i, (0, B_PAD - B))
    neigh_p = jnp.pad(neigh_i, ((0, B_PAD - B), (0, 0))).reshape(-1)
    comb = _sc_gather(features, nodes_p, neigh_p)
    out = _tc_matmul(comb, weight)
    return out[:B]


# speedup vs baseline: 4.3950x; 2.0437x over previous
import functools

import jax
import jax.numpy as jnp
from jax import lax
from jax.experimental import pallas as pl
from jax.experimental.pallas import tpu as pltpu
from jax.experimental.pallas import tpu_sc as plsc

B = 25000
D = 128
S = 20
NV = D // 16
NS = 16
BPW = 1568
B_PAD = NS * BPW
Q = 16
NCH = BPW // Q
G = 4
GROWS = Q * S // G

LAST_VALID = B - (NS - 1) * BPW
LAST_VALID_N = LAST_VALID * S

_mesh = plsc.VectorSubcoreMesh(core_axis_name="c", subcore_axis_name="s")


@functools.partial(
    pl.kernel,
    out_type=jax.ShapeDtypeStruct((B_PAD, 2 * D), jnp.float32),
    mesh=_mesh,
    scratch_types=[
        pltpu.VMEM((BPW,), jnp.int32),
        pltpu.VMEM((BPW * S,), jnp.int32),
        pltpu.VMEM((Q * S, D), jnp.float32),
        pltpu.VMEM((Q * S, D), jnp.float32),
        pltpu.VMEM((Q, D), jnp.float32),
        pltpu.VMEM((Q, D), jnp.float32),
        pltpu.VMEM((Q, 2 * D), jnp.float32),
        pltpu.VMEM((Q, 2 * D), jnp.float32),
        pltpu.SemaphoreType.DMA,
        pltpu.SemaphoreType.DMA,
        pltpu.SemaphoreType.DMA,
        pltpu.SemaphoreType.DMA,
    ],
)
def _sc_gather(feat_hbm, nodes_hbm, neigh_hbm, comb_hbm,
               nodes_v, nidx_v, rows0, rows1, srows0, srows1,
               stage0, stage1, gsem0, gsem1, osem0, osem1):
    cid = lax.axis_index("c")
    sid = lax.axis_index("s")

    @pl.when(cid == 0)
    def _run():
        base = sid * BPW

        @pl.when(sid < NS - 1)
        def _load_idx_full():
            pltpu.sync_copy(nodes_hbm.at[pl.ds(base, BPW)], nodes_v)
            pltpu.sync_copy(neigh_hbm.at[pl.ds(base * S, BPW * S)], nidx_v)

        @pl.when(sid == NS - 1)
        def _load_idx_tail():
            zi = jnp.zeros((16,), jnp.int32)

            def zn(i, carry):
                nidx_v[pl.ds(i * 16, 16)] = zi
                return carry

            lax.fori_loop(0, BPW * S // 16, zn, 0)

            def zs(i, carry):
                nodes_v[pl.ds(i * 16, 16)] = zi
                return carry

            lax.fori_loop(0, BPW // 16, zs, 0)
            pltpu.sync_copy(nodes_hbm.at[pl.ds(base, LAST_VALID)],
                            nodes_v.at[pl.ds(0, LAST_VALID)])
            pltpu.sync_copy(neigh_hbm.at[pl.ds(base * S, LAST_VALID_N)],
                            nidx_v.at[pl.ds(0, LAST_VALID_N)])

        rows = (rows0, rows1)
        srows = (srows0, srows1)
        stage = (stage0, stage1)
        gsem = (gsem0, gsem1)
        osem = (osem0, osem1)

        def issue(c, p):
            for g in range(G):
                pltpu.async_copy(
                    feat_hbm.at[
                        nidx_v.at[pl.ds(c * (Q * S) + g * GROWS, GROWS)]],
                    rows[p].at[pl.ds(g * GROWS, GROWS)],
                    gsem[p])
            pltpu.async_copy(feat_hbm.at[nodes_v.at[pl.ds(c * Q, Q)]],
                             srows[p], gsem[p])

        def wait_gathers(p):
            for g in range(G):
                pltpu.make_async_copy(
                    feat_hbm.at[pl.ds(0, GROWS)],
                    rows[p].at[pl.ds(g * GROWS, GROWS)],
                    gsem[p]).wait()
            pltpu.make_async_copy(feat_hbm.at[pl.ds(0, Q)], srows[p],
                                  gsem[p]).wait()

        def accum(p):
            r = rows[p]
            sr = srows[p]
            st = stage[p]

            def qbody(q, carry):
                for v in range(NV):
                    sl = pl.ds(v * 16, 16)
                    st[q, sl] = sr[q, sl]
                accs = [r[q * S, pl.ds(v * 16, 16)] for v in range(NV)]
                for s in range(1, S):
                    for v in range(NV):
                        accs[v] = accs[v] + r[q * S + s, pl.ds(v * 16, 16)]
                for v in range(NV):
                    st[q, pl.ds(D + v * 16, 16)] = accs[v] * (1.0 / S)
                return carry

            lax.fori_loop(0, Q, qbody, 0)

        issue(0, 0)
        issue(1, 1)

        def tbody(t, carry):
            for p in (0, 1):
                c = 2 * t + p
                wait_gathers(p)

                @pl.when(t > 0)
                def _wait_out():
                    pltpu.make_async_copy(stage[p], comb_hbm.at[pl.ds(0, Q)],
                                          osem[p]).wait()

                accum(p)
                pltpu.async_copy(stage[p],
                                 comb_hbm.at[pl.ds(base + c * Q, Q)],
                                 osem[p])
                cn = c + 2

                @pl.when(cn < NCH)
                def _issue_next():
                    issue(cn, p)
            return carry

        lax.fori_loop(0, NCH // 2, tbody, 0)
        for p in (0, 1):
            pltpu.make_async_copy(stage[p], comb_hbm.at[pl.ds(0, Q)],
                                  osem[p]).wait()


BLK = 1000


def _mm_body(x_ref, w_ref, o_ref):
    o_ref[...] = jnp.maximum(
        jnp.dot(x_ref[...], w_ref[...], preferred_element_type=jnp.float32),
        0.0)


def _tc_matmul(comb, w):
    return pl.pallas_call(
        _mm_body,
        grid=(B // BLK,),
        in_specs=[
            pl.BlockSpec((BLK, 2 * D), lambda i: (i, 0)),
            pl.BlockSpec((2 * D, D), lambda i: (0, 0)),
        ],
        out_specs=pl.BlockSpec((BLK, D), lambda i: (i, 0)),
        out_shape=jax.ShapeDtypeStruct((B, D), jnp.float32),
    )(comb, w)


@jax.jit
def kernel(nodes, neigh_idx, features, weight):
    nodes_i = nodes.astype(jnp.int32)
    neigh_i = neigh_idx.astype(jnp.int32).reshape(-1)
    comb = _sc_gather(features, nodes_i, neigh_i)
    return _tc_matmul(comb, weight)
